# baseline (speedup 1.0000x reference)
import functools

import jax
import jax.numpy as jnp
from jax import lax
from jax.experimental import pallas as pl
from jax.experimental.pallas import tpu as pltpu
from jax.experimental.pallas import tpu_sc as plsc

_N = 10000
_E = 320000
_D = 128
_NCORE = 2
_NSUB = 16
_NW = _NCORE * _NSUB
_NPAD = 10112
_CPT = _D // _NSUB
_CHE = 512
_ESC = _E // _NCORE
_NCHE = -(-_ESC // _CHE)
_NCHEP = _NCHE + 2
_ACCW = _NPAD * _CPT


def _seg_sum_sc(hT, edges_sc):
    mesh = plsc.VectorSubcoreMesh(core_axis_name="c", subcore_axis_name="s")

    @functools.partial(
        pl.kernel,
        mesh=mesh,
        out_type=jax.ShapeDtypeStruct((_NCORE, _NSUB, _ACCW), jnp.float32),
        scratch_types=[
            pltpu.VMEM((2, _CHE), jnp.int32),
            pltpu.VMEM((2, _CHE), jnp.int32),
            pltpu.VMEM((2, _CHE, _CPT), jnp.float32),
            pltpu.VMEM((_ACCW,), jnp.float32),
            pltpu.SemaphoreType.DMA((2,)),
            pltpu.SemaphoreType.DMA((2,)),
        ],
        compiler_params=pltpu.CompilerParams(needs_layout_passes=False,
                                             use_tc_tiling_on_sc=False),
    )
    def seg_kernel(hT_hbm, edges_hbm, out_hbm, pkb, sidxb, stage, acc,
                   psems, gsems):
        cid = lax.axis_index("c")
        sid = lax.axis_index("s")
        iota16 = lax.iota(jnp.int32, 16)

        def zacc(i, carry):
            for u in range(8):
                acc[pl.ds(128 * i + 16 * u, 16)] = jnp.zeros((16,),
                                                             jnp.float32)
            return carry

        lax.fori_loop(0, _ACCW // 128, zacc, 0)

        def unpack_src(par):
            @plsc.parallel_loop(0, _CHE // 16, unroll=4)
            def grp(g):
                v = pkb[par, pl.ds(16 * g, 16)]
                sidxb[par, pl.ds(16 * g, 16)] = v >> 14

        def compute(par):
            @plsc.parallel_loop(0, _CHE // 16, unroll=4)
            def grp(g):
                vpk = pkb[par, pl.ds(16 * g, 16)]
                d8 = (vpk & 16383) * 8
                rowv = 16 * g + iota16
                for c in range(_CPT):
                    colv = jnp.full((16,), c, jnp.int32)
                    v = plsc.load_gather(stage.at[par], [rowv, colv])
                    plsc.addupdate_scatter(acc, [d8 + c], v)

        pltpu.async_copy(edges_hbm.at[cid, 0], pkb.at[0], psems.at[0])
        pltpu.async_copy(edges_hbm.at[cid, 1], pkb.at[1], psems.at[1])
        pltpu.make_async_copy(edges_hbm.at[cid, 0], pkb.at[0],
                              psems.at[0]).wait()
        unpack_src(0)
        pltpu.async_copy(hT_hbm.at[sid].at[sidxb.at[0]], stage.at[0],
                         gsems.at[0])

        def body(j, carry):
            p = lax.rem(j, 2)
            pn = lax.rem(j + 1, 2)
            pltpu.make_async_copy(edges_hbm.at[cid, 0], pkb.at[pn],
                                  psems.at[pn]).wait()
            unpack_src(pn)
            pltpu.async_copy(hT_hbm.at[sid].at[sidxb.at[pn]], stage.at[pn],
                             gsems.at[pn])
            pltpu.make_async_copy(hT_hbm.at[sid].at[sidxb.at[p]],
                                  stage.at[p], gsems.at[p]).wait()
            compute(p)
            pltpu.async_copy(edges_hbm.at[cid, j + 2], pkb.at[p],
                             psems.at[p])
            return carry

        lax.fori_loop(0, _NCHE, body, 0)
        pltpu.make_async_copy(hT_hbm.at[sid].at[sidxb.at[0]],
                              stage.at[_NCHE % 2], gsems.at[_NCHE % 2]).wait()
        pltpu.make_async_copy(edges_hbm.at[cid, 0], pkb.at[(_NCHE + 1) % 2],
                              psems.at[(_NCHE + 1) % 2]).wait()

        pltpu.sync_copy(acc, out_hbm.at[cid, sid])

    return seg_kernel(hT, edges_sc)


_BM = 2000


def _full(shape):
    return pl.BlockSpec(shape, lambda i: (0, 0))


def _pre_tc(x, w, b):
    def body(x_ref, w_ref, b_ref, o_ref):
        o_ref[...] = (
            jnp.dot(x_ref[...], w_ref[...], preferred_element_type=jnp.float32)
            + b_ref[...]
        )

    return pl.pallas_call(
        body,
        grid=(_N // _BM,),
        in_specs=[
            pl.BlockSpec((_BM, _D), lambda i: (i, 0)),
            _full((_D, _D)),
            _full((1, _D)),
        ],
        out_specs=pl.BlockSpec((_BM, _D), lambda i: (i, 0)),
        out_shape=jax.ShapeDtypeStruct((_N, _D), jnp.float32),
    )(x, w, b.reshape(1, _D))


def _mlp_tc(h, agg, w1, b1, w2, b2):
    def body(h_ref, a0_ref, a1_ref, w1_ref, b1_ref, w2_ref, b2_ref, o_ref):
        z = h_ref[...] + a0_ref[...] + a1_ref[...]
        z = jnp.maximum(
            jnp.dot(z, w1_ref[...], preferred_element_type=jnp.float32)
            + b1_ref[...],
            0.0,
        )
        z = (
            jnp.dot(z, w2_ref[...], preferred_element_type=jnp.float32)
            + b2_ref[...]
        )
        o_ref[...] = jnp.maximum(z, 0.0)

    return pl.pallas_call(
        body,
        grid=(_N // _BM,),
        in_specs=[
            pl.BlockSpec((_BM, _D), lambda i: (i, 0)),
            pl.BlockSpec((_BM, _D), lambda i: (i, 0)),
            pl.BlockSpec((_BM, _D), lambda i: (i, 0)),
            _full((_D, _D)),
            _full((1, _D)),
            _full((_D, _D)),
            _full((1, _D)),
        ],
        out_specs=pl.BlockSpec((_BM, _D), lambda i: (i, 0)),
        out_shape=jax.ShapeDtypeStruct((_N, _D), jnp.float32),
    )(h, agg[0], agg[1], w1, b1.reshape(1, _D), w2, b2.reshape(1, _D))


def _post_tc(h, wp, bp, wr_pad, br_pad):
    def body(h_ref, wp_ref, bp_ref, wr_ref, br_ref, o_ref):
        t = jnp.maximum(
            jnp.dot(h_ref[...], wp_ref[...], preferred_element_type=jnp.float32)
            + bp_ref[...],
            0.0,
        )
        z = (
            jnp.dot(t, wr_ref[...], preferred_element_type=jnp.float32)
            + br_ref[...]
        )
        m = jnp.max(z, axis=1, keepdims=True)
        lse = jnp.log(jnp.sum(jnp.exp(z - m), axis=1, keepdims=True)) + m
        o_ref[...] = z - lse

    return pl.pallas_call(
        body,
        grid=(_N // _BM,),
        in_specs=[
            pl.BlockSpec((_BM, _D), lambda i: (i, 0)),
            _full((_D, _D)),
            _full((1, _D)),
            _full((_D, _D)),
            _full((1, _D)),
        ],
        out_specs=pl.BlockSpec((_BM, _D), lambda i: (i, 0)),
        out_shape=jax.ShapeDtypeStruct((_N, _D), jnp.float32),
    )(h, wp, bp.reshape(1, _D), wr_pad, br_pad)


def kernel(x, edge_index, W_pre, b_pre, W1s, b1s, W2s, b2s, W_post, b_post,
           W_ro, b_ro):
    src = edge_index[0]
    dst = edge_index[1]
    npad = _NCHEP * _CHE - _ESC
    packed = jnp.bitwise_or(jnp.left_shift(src, 14), dst)
    edges_sc = jnp.concatenate(
        [packed.reshape(_NCORE, _ESC), jnp.full((_NCORE, npad), _N,
                                                jnp.int32)],
        axis=1).reshape(_NCORE, _NCHEP, _CHE)

    h = _pre_tc(x, W_pre, b_pre)
    for l in range(3):
        hT = h.reshape(_N, _NSUB, _CPT).transpose(1, 0, 2)
        agg_sh = _seg_sum_sc(hT, edges_sc)
        agg = (agg_sh.reshape(_NCORE, _NSUB, _NPAD, _CPT)
               .transpose(0, 2, 1, 3).reshape(_NCORE, _NPAD, _D))
        h = _mlp_tc(h, agg, W1s[l], b1s[l], W2s[l], b2s[l])

    nclass = W_ro.shape[1]
    wr_pad = jnp.zeros((_D, _D), jnp.float32).at[:, :nclass].set(W_ro)
    br_pad = jnp.full((1, _D), -1e30, jnp.float32).at[0, :nclass].set(b_ro)
    out = _post_tc(h, W_post, b_post, wr_pad, br_pad)[:, :nclass]
    return (out, h, h)


# speedup vs baseline: 1.3893x; 1.3893x over previous
import functools

import jax
import jax.numpy as jnp
from jax import lax
from jax.experimental import pallas as pl
from jax.experimental.pallas import tpu as pltpu
from jax.experimental.pallas import tpu_sc as plsc

_N = 10000
_E = 320000
_D = 128
_NCORE = 2
_NSUB = 16
_NW = _NCORE * _NSUB
_CH = 128
_NCH = 80
_EPW = _CH * _NCH
_EPAD = _NW * _EPW
_NPAD = 10112
_RPT = _NPAD // _NSUB


def _seg_sum_sc(h, src_r, dst_r):
    mesh = plsc.VectorSubcoreMesh(core_axis_name="c", subcore_axis_name="s")

    @functools.partial(
        pl.kernel,
        mesh=mesh,
        out_type=jax.ShapeDtypeStruct((_NCORE, _NPAD, _D), jnp.float32),
        scratch_types=[
            pltpu.VMEM((_NCH, _CH), jnp.int32),
            pltpu.VMEM((_NCH, _CH), jnp.int32),
            pltpu.VMEM((_CH, _D), jnp.float32),
            pltpu.VMEM_SHARED((_NPAD, _D), jnp.float32),
            pltpu.SemaphoreType.DMA,
        ],
    )
    def seg_kernel(h_hbm, src_hbm, dst_hbm, out_hbm, srcv, dstv, rows, acc, gsem):
        cid = lax.axis_index("c")
        sid = lax.axis_index("s")
        wid = sid * _NCORE + cid

        def zrow(i, carry):
            for j in range(_D // 16):
                rows[i, pl.ds(16 * j, 16)] = jnp.zeros((16,), jnp.float32)
            return carry

        lax.fori_loop(0, _CH, zrow, 0)
        base = sid * _RPT
        for k in range(_RPT // _CH):
            pltpu.sync_copy(rows, acc.at[pl.ds(base + k * _CH, _CH)])
        rem = _RPT % _CH
        if rem:
            pltpu.sync_copy(rows.at[pl.ds(0, rem)],
                            acc.at[pl.ds(base + (_RPT // _CH) * _CH, rem)])
        plsc.subcore_barrier()

        pltpu.sync_copy(src_hbm.at[wid], srcv)
        pltpu.sync_copy(dst_hbm.at[wid], dstv)

        def body(j, carry):
            pltpu.async_copy(h_hbm.at[srcv.at[j]], rows, gsem).wait()
            pltpu.sync_copy(rows, acc.at[dstv.at[j]], add=True)
            return carry

        lax.fori_loop(0, _NCH, body, 0)
        plsc.subcore_barrier()

        pltpu.sync_copy(acc.at[pl.ds(base, _RPT)],
                        out_hbm.at[cid, pl.ds(base, _RPT)])

    return seg_kernel(h, src_r, dst_r)


_BM = 2000


def _full(shape):
    return pl.BlockSpec(shape, lambda i: (0, 0))


def _pre_tc(x, w, b):
    def body(x_ref, w_ref, b_ref, o_ref):
        o_ref[...] = (
            jnp.dot(x_ref[...], w_ref[...], preferred_element_type=jnp.float32)
            + b_ref[...]
        )

    return pl.pallas_call(
        body,
        grid=(_N // _BM,),
        in_specs=[
            pl.BlockSpec((_BM, _D), lambda i: (i, 0)),
            _full((_D, _D)),
            _full((1, _D)),
        ],
        out_specs=pl.BlockSpec((_BM, _D), lambda i: (i, 0)),
        out_shape=jax.ShapeDtypeStruct((_N, _D), jnp.float32),
    )(x, w, b.reshape(1, _D))


def _mlp_tc(h, agg, w1, b1, w2, b2):
    def body(h_ref, a0_ref, a1_ref, w1_ref, b1_ref, w2_ref, b2_ref, o_ref):
        z = h_ref[...] + a0_ref[...] + a1_ref[...]
        z = jnp.maximum(
            jnp.dot(z, w1_ref[...], preferred_element_type=jnp.float32)
            + b1_ref[...],
            0.0,
        )
        z = (
            jnp.dot(z, w2_ref[...], preferred_element_type=jnp.float32)
            + b2_ref[...]
        )
        o_ref[...] = jnp.maximum(z, 0.0)

    return pl.pallas_call(
        body,
        grid=(_N // _BM,),
        in_specs=[
            pl.BlockSpec((_BM, _D), lambda i: (i, 0)),
            pl.BlockSpec((_BM, _D), lambda i: (i, 0)),
            pl.BlockSpec((_BM, _D), lambda i: (i, 0)),
            _full((_D, _D)),
            _full((1, _D)),
            _full((_D, _D)),
            _full((1, _D)),
        ],
        out_specs=pl.BlockSpec((_BM, _D), lambda i: (i, 0)),
        out_shape=jax.ShapeDtypeStruct((_N, _D), jnp.float32),
    )(h, agg[0], agg[1], w1, b1.reshape(1, _D), w2, b2.reshape(1, _D))


def _post_tc(h, wp, bp, wr_pad, br_pad):
    def body(h_ref, wp_ref, bp_ref, wr_ref, br_ref, o_ref):
        t = jnp.maximum(
            jnp.dot(h_ref[...], wp_ref[...], preferred_element_type=jnp.float32)
            + bp_ref[...],
            0.0,
        )
        z = (
            jnp.dot(t, wr_ref[...], preferred_element_type=jnp.float32)
            + br_ref[...]
        )
        m = jnp.max(z, axis=1, keepdims=True)
        lse = jnp.log(jnp.sum(jnp.exp(z - m), axis=1, keepdims=True)) + m
        o_ref[...] = z - lse

    return pl.pallas_call(
        body,
        grid=(_N // _BM,),
        in_specs=[
            pl.BlockSpec((_BM, _D), lambda i: (i, 0)),
            _full((_D, _D)),
            _full((1, _D)),
            _full((_D, _D)),
            _full((1, _D)),
        ],
        out_specs=pl.BlockSpec((_BM, _D), lambda i: (i, 0)),
        out_shape=jax.ShapeDtypeStruct((_N, _D), jnp.float32),
    )(h, wp, bp.reshape(1, _D), wr_pad, br_pad)


def kernel(x, edge_index, W_pre, b_pre, W1s, b1s, W2s, b2s, W_post, b_post,
           W_ro, b_ro):
    src = edge_index[0]
    dst = edge_index[1]
    npad = _EPAD - _E
    src_r = jnp.concatenate([src, jnp.zeros((npad,), jnp.int32)]).reshape(
        _NW, _NCH, _CH)
    dst_r = jnp.concatenate([dst, jnp.full((npad,), _N, jnp.int32)]).reshape(
        _NW, _NCH, _CH)

    h = _pre_tc(x, W_pre, b_pre)
    for l in range(3):
        agg = _seg_sum_sc(h, src_r, dst_r)
        h = _mlp_tc(h, agg, W1s[l], b1s[l], W2s[l], b2s[l])

    nclass = W_ro.shape[1]
    wr_pad = jnp.zeros((_D, _D), jnp.float32).at[:, :nclass].set(W_ro)
    br_pad = jnp.full((1, _D), -1e30, jnp.float32).at[0, :nclass].set(b_ro)
    out = _post_tc(h, W_post, b_post, wr_pad, br_pad)[:, :nclass]
    return (out, h, h)
